# baseline (speedup 1.0000x reference)
import functools

import jax
import jax.numpy as jnp
from jax import lax
from jax.experimental import pallas as pl
from jax.experimental.pallas import tpu as pltpu
from jax.experimental.pallas import tpu_sc as plsc

_DIM = 32
_NC = 2
_NS = 16
_NW = _NC * _NS
_CHUNK = 800
_NBUF = 4

_VOCAB = 1000000
_TBLK = 2048
_Q = _TBLK // 4
_NBLK = (_VOCAB + _TBLK - 1) // _TBLK
_NLINES = _NBLK * _Q

_B = 4096
_J = 200


def _repack_body(wt_ref, w4_ref):
  blk = wt_ref[...]
  t = jnp.transpose(blk)
  w4_ref[...] = jnp.concatenate(
      [t[q * _Q:(q + 1) * _Q, :] for q in range(4)], axis=1)


def _repack_table(wt):
  return pl.pallas_call(
      _repack_body,
      grid=(_NBLK,),
      in_specs=[pl.BlockSpec((_DIM, _TBLK), lambda i: (0, i))],
      out_specs=pl.BlockSpec((_Q, 128), lambda i: (i, 0)),
      out_shape=jax.ShapeDtypeStruct((_NLINES, 128), jnp.float32),
  )(wt)


_mesh = plsc.VectorSubcoreMesh(core_axis_name="c", subcore_axis_name="s")

_B_PER_W = _B * _J // _NW
_N_CHUNKS = _B_PER_W // _CHUNK
_ROWS_PER_CHUNK = _CHUNK // _J


@functools.partial(
    pl.kernel,
    mesh=_mesh,
    out_type=jax.ShapeDtypeStruct((_B * _J, _DIM), jnp.float32),
    scratch_types=[
        pltpu.VMEM((_B_PER_W,), jnp.int32),
        pltpu.VMEM((_NBUF, _CHUNK, _DIM), jnp.float32),
        pltpu.SemaphoreType.DMA((_NBUF,)),
        pltpu.SemaphoreType.DMA((_NBUF,)),
    ],
    compiler_params=pltpu.CompilerParams(use_tc_tiling_on_sc=False),
)
def _emb(idx_hbm, table_hbm, out_hbm, idx_full, bufs, gsems, wsems):
  wid = lax.axis_index("s") * _NC + lax.axis_index("c")
  base = wid * _B_PER_W
  row0 = wid * (_B // _NW)
  pltpu.sync_copy(idx_hbm.at[pl.ds(base, _B_PER_W)], idx_full)

  def remap(k, _):
    v = idx_full[pl.ds(k * 16, 16)]
    p = (v & -2048) | ((v & 511) << 2) | ((v >> 9) & 3)
    idx_full[pl.ds(k * 16, 16)] = p
    return 0

  lax.fori_loop(0, _B_PER_W // 16, remap, 0)

  def gather(i, b):
    return pltpu.make_async_copy(
        table_hbm.at[idx_full.at[pl.ds(i * _CHUNK, _CHUNK)]],
        bufs.at[b],
        gsems.at[b],
    )

  def wb(i, b):
    return pltpu.make_async_copy(
        bufs.at[b],
        out_hbm.at[pl.ds(base + i * _CHUNK, _CHUNK)],
        wsems.at[b],
    )

  def slot(i, b, do_a, do_b):
    if do_a:
      wb(i - 1, (b - 1) % _NBUF).wait()
    if do_b:
      gather(i + _NBUF - 1, (b - 1) % _NBUF).start()
    gather(i, b).wait()
    wb(i, b).start()

  for b in range(_NBUF):
    gather(b, b).start()

  slot(0, 0, False, False)
  for b in range(1, _NBUF):
    slot(b, b, True, True)

  def body(g, _):
    i0 = g * _NBUF
    for b in range(_NBUF):
      slot(i0 + b, b, True, True)
    return 0

  lax.fori_loop(1, _N_CHUNKS // _NBUF - 1, body, 0)

  i0 = _N_CHUNKS - _NBUF
  slot(i0, 0, True, True)
  for b in range(1, _NBUF):
    slot(i0 + b, b, True, False)

  wb(_N_CHUNKS - 1, _NBUF - 1).wait()


def kernel(x, weight):
  wt = jnp.transpose(weight)
  w4 = _repack_table(wt)
  w_row = w4.reshape(_NLINES * 4, _DIM)
  out = _emb(x.reshape(-1), w_row)
  return out.reshape(_B, _J, _DIM)


# speedup vs baseline: 1.0002x; 1.0002x over previous
import functools

import jax
import jax.numpy as jnp
from jax import lax
from jax.experimental import pallas as pl
from jax.experimental.pallas import tpu as pltpu
from jax.experimental.pallas import tpu_sc as plsc

_DIM = 32
_NC = 2
_NS = 16
_NW = _NC * _NS
_CHUNK = 800
_NBUF = 4

_VOCAB = 1000000
_TBLK = 2048
_Q = _TBLK // 4
_NBLK = (_VOCAB + _TBLK - 1) // _TBLK
_NLINES = _NBLK * _Q

_B = 4096
_J = 200


def _repack_body(wt_ref, w4_ref):
  blk = wt_ref[...]
  eye = jnp.eye(_DIM, dtype=jnp.float32)
  t = lax.dot_general(blk, eye, (((0,), (0,)), ((), ())),
                      preferred_element_type=jnp.float32)
  w4_ref[...] = jnp.concatenate(
      [t[q * _Q:(q + 1) * _Q, :] for q in range(4)], axis=1)


def _repack_table(wt):
  return pl.pallas_call(
      _repack_body,
      grid=(_NBLK,),
      in_specs=[pl.BlockSpec((_DIM, _TBLK), lambda i: (0, i))],
      out_specs=pl.BlockSpec((_Q, 128), lambda i: (i, 0)),
      out_shape=jax.ShapeDtypeStruct((_NLINES, 128), jnp.float32),
      compiler_params=pltpu.CompilerParams(fuse_transposed_lhs_in_matmul=True),
  )(wt)


_mesh = plsc.VectorSubcoreMesh(core_axis_name="c", subcore_axis_name="s")

_B_PER_W = _B * _J // _NW
_N_CHUNKS = _B_PER_W // _CHUNK
_ROWS_PER_CHUNK = _CHUNK // _J


@functools.partial(
    pl.kernel,
    mesh=_mesh,
    out_type=jax.ShapeDtypeStruct((_B * _J, _DIM), jnp.float32),
    scratch_types=[
        pltpu.VMEM((_B_PER_W,), jnp.int32),
        pltpu.VMEM((_NBUF, _CHUNK, _DIM), jnp.float32),
        pltpu.SemaphoreType.DMA((_NBUF,)),
        pltpu.SemaphoreType.DMA((_NBUF,)),
    ],
    compiler_params=pltpu.CompilerParams(use_tc_tiling_on_sc=False),
)
def _emb(idx_hbm, table_hbm, out_hbm, idx_full, bufs, gsems, wsems):
  wid = lax.axis_index("s") * _NC + lax.axis_index("c")
  base = wid * _B_PER_W
  row0 = wid * (_B // _NW)
  pltpu.sync_copy(idx_hbm.at[pl.ds(base, _B_PER_W)], idx_full)

  def remap(k, _):
    v = idx_full[pl.ds(k * 16, 16)]
    p = (v & -2048) | ((v & 511) << 2) | ((v >> 9) & 3)
    idx_full[pl.ds(k * 16, 16)] = p
    return 0

  lax.fori_loop(0, _B_PER_W // 16, remap, 0)

  def gather(i, b):
    return pltpu.make_async_copy(
        table_hbm.at[idx_full.at[pl.ds(i * _CHUNK, _CHUNK)]],
        bufs.at[b],
        gsems.at[b],
    )

  def wb(i, b):
    return pltpu.make_async_copy(
        bufs.at[b],
        out_hbm.at[pl.ds(base + i * _CHUNK, _CHUNK)],
        wsems.at[b],
    )

  def slot(i, b, do_a, do_b):
    if do_a:
      wb(i - 1, (b - 1) % _NBUF).wait()
    if do_b:
      gather(i + _NBUF - 1, (b - 1) % _NBUF).start()
    gather(i, b).wait()
    wb(i, b).start()

  for b in range(_NBUF):
    gather(b, b).start()

  slot(0, 0, False, False)
  for b in range(1, _NBUF):
    slot(b, b, True, True)

  def body(g, _):
    i0 = g * _NBUF
    for b in range(_NBUF):
      slot(i0 + b, b, True, True)
    return 0

  lax.fori_loop(1, _N_CHUNKS // _NBUF - 1, body, 0)

  i0 = _N_CHUNKS - _NBUF
  slot(i0, 0, True, True)
  for b in range(1, _NBUF):
    slot(i0 + b, b, True, False)

  wb(_N_CHUNKS - 1, _NBUF - 1).wait()


def kernel(x, weight):
  wt = jnp.transpose(weight)
  w4 = _repack_table(wt)
  w_row = w4.reshape(_NLINES * 4, _DIM)
  out = _emb(x.reshape(-1), w_row)
  return out.reshape(_B, _J, _DIM)


# speedup vs baseline: 1.2114x; 1.2112x over previous
import functools

import jax
import jax.numpy as jnp
from jax import lax
from jax.experimental import pallas as pl
from jax.experimental.pallas import tpu as pltpu
from jax.experimental.pallas import tpu_sc as plsc

_DIM = 32
_NC = 2
_NS = 16
_NW = _NC * _NS
_CHUNK = 800
_NBUF = 4

_VOCAB = 1000000
_TBLK = 8192
_Q = _TBLK // 4
_NBLK = (_VOCAB + _TBLK - 1) // _TBLK
_NLINES = _NBLK * _Q

_B = 4096
_J = 200


def _repack_body(wt_ref, w4_ref):
  blk = wt_ref[...]
  t = jnp.transpose(blk)
  w4_ref[...] = jnp.concatenate(
      [t[q * _Q:(q + 1) * _Q, :] for q in range(4)], axis=1)


def _repack_table(wt):
  return pl.pallas_call(
      _repack_body,
      grid=(_NBLK,),
      in_specs=[pl.BlockSpec((_DIM, _TBLK), lambda i: (0, i))],
      out_specs=pl.BlockSpec((_Q, 128), lambda i: (i, 0)),
      out_shape=jax.ShapeDtypeStruct((_NLINES, 128), jnp.float32),
  )(wt)


_mesh = plsc.VectorSubcoreMesh(core_axis_name="c", subcore_axis_name="s")

_B_PER_W = _B * _J // _NW
_N_CHUNKS = _B_PER_W // _CHUNK
_ROWS_PER_CHUNK = _CHUNK // _J


@functools.partial(
    pl.kernel,
    mesh=_mesh,
    out_type=jax.ShapeDtypeStruct((_B * _J, _DIM), jnp.float32),
    scratch_types=[
        pltpu.VMEM((_B_PER_W,), jnp.int32),
        pltpu.VMEM((_NBUF, _CHUNK, _DIM), jnp.float32),
        pltpu.SemaphoreType.DMA((_NBUF,)),
        pltpu.SemaphoreType.DMA((_NBUF,)),
    ],
    compiler_params=pltpu.CompilerParams(use_tc_tiling_on_sc=False),
)
def _emb(idx_hbm, table_hbm, out_hbm, idx_full, bufs, gsems, wsems):
  wid = lax.axis_index("s") * _NC + lax.axis_index("c")
  base = wid * _B_PER_W
  row0 = wid * (_B // _NW)
  pltpu.sync_copy(idx_hbm.at[pl.ds(base, _B_PER_W)], idx_full)

  qshift = _Q.bit_length() - 1

  def remap(k, _):
    v = idx_full[pl.ds(k * 16, 16)]
    p = (v & -_TBLK) | ((v & (_Q - 1)) << 2) | ((v >> qshift) & 3)
    idx_full[pl.ds(k * 16, 16)] = p
    return 0

  lax.fori_loop(0, _B_PER_W // 16, remap, 0)

  def gather(i, b):
    return pltpu.make_async_copy(
        table_hbm.at[idx_full.at[pl.ds(i * _CHUNK, _CHUNK)]],
        bufs.at[b],
        gsems.at[b],
    )

  def wb(i, b):
    return pltpu.make_async_copy(
        bufs.at[b],
        out_hbm.at[pl.ds(base + i * _CHUNK, _CHUNK)],
        wsems.at[b],
    )

  def slot(i, b, do_a, do_b):
    if do_a:
      wb(i - 1, (b - 1) % _NBUF).wait()
    if do_b:
      gather(i + _NBUF - 1, (b - 1) % _NBUF).start()
    gather(i, b).wait()
    wb(i, b).start()

  for b in range(_NBUF):
    gather(b, b).start()

  slot(0, 0, False, False)
  for b in range(1, _NBUF):
    slot(b, b, True, True)

  def body(g, _):
    i0 = g * _NBUF
    for b in range(_NBUF):
      slot(i0 + b, b, True, True)
    return 0

  lax.fori_loop(1, _N_CHUNKS // _NBUF - 1, body, 0)

  i0 = _N_CHUNKS - _NBUF
  slot(i0, 0, True, True)
  for b in range(1, _NBUF):
    slot(i0 + b, b, True, False)

  wb(_N_CHUNKS - 1, _NBUF - 1).wait()


def kernel(x, weight):
  wt = jnp.transpose(weight)
  w4 = _repack_table(wt)
  w_row = w4.reshape(_NLINES * 4, _DIM)
  out = _emb(x.reshape(-1), w_row)
  return out.reshape(_B, _J, _DIM)


# speedup vs baseline: 1.2179x; 1.0054x over previous
import functools

import jax
import jax.numpy as jnp
from jax import lax
from jax.experimental import pallas as pl
from jax.experimental.pallas import tpu as pltpu
from jax.experimental.pallas import tpu_sc as plsc

_DIM = 32
_NC = 2
_NS = 16
_NW = _NC * _NS
_CHUNK = 800
_NBUF = 4

_VOCAB = 1000000
_TBLK = 16384
_Q = _TBLK // 4
_NBLK = (_VOCAB + _TBLK - 1) // _TBLK
_NLINES = _NBLK * _Q

_B = 4096
_J = 200


def _repack_body(wt_ref, w4_ref):
  blk = wt_ref[...]
  t = jnp.transpose(blk)
  w4_ref[...] = jnp.concatenate(
      [t[q * _Q:(q + 1) * _Q, :] for q in range(4)], axis=1)


def _repack_table(wt):
  return pl.pallas_call(
      _repack_body,
      grid=(_NBLK,),
      in_specs=[pl.BlockSpec((_DIM, _TBLK), lambda i: (0, i))],
      out_specs=pl.BlockSpec((_Q, 128), lambda i: (i, 0)),
      out_shape=jax.ShapeDtypeStruct((_NLINES, 128), jnp.float32),
  )(wt)


_mesh = plsc.VectorSubcoreMesh(core_axis_name="c", subcore_axis_name="s")

_B_PER_W = _B * _J // _NW
_N_CHUNKS = _B_PER_W // _CHUNK
_ROWS_PER_CHUNK = _CHUNK // _J


@functools.partial(
    pl.kernel,
    mesh=_mesh,
    out_type=jax.ShapeDtypeStruct((_B * _J, _DIM), jnp.float32),
    scratch_types=[
        pltpu.VMEM((_B_PER_W,), jnp.int32),
        pltpu.VMEM((_NBUF, _CHUNK, _DIM), jnp.float32),
        pltpu.SemaphoreType.DMA((_NBUF,)),
        pltpu.SemaphoreType.DMA((_NBUF,)),
    ],
    compiler_params=pltpu.CompilerParams(use_tc_tiling_on_sc=False),
)
def _emb(idx_hbm, table_hbm, out_hbm, idx_full, bufs, gsems, wsems):
  wid = lax.axis_index("s") * _NC + lax.axis_index("c")
  base = wid * _B_PER_W
  row0 = wid * (_B // _NW)
  pltpu.sync_copy(idx_hbm.at[pl.ds(base, _B_PER_W)], idx_full)

  qshift = _Q.bit_length() - 1

  def remap(k, _):
    v = idx_full[pl.ds(k * 16, 16)]
    p = (v & -_TBLK) | ((v & (_Q - 1)) << 2) | ((v >> qshift) & 3)
    idx_full[pl.ds(k * 16, 16)] = p
    return 0

  lax.fori_loop(0, _B_PER_W // 16, remap, 0)

  def gather(i, b):
    return pltpu.make_async_copy(
        table_hbm.at[idx_full.at[pl.ds(i * _CHUNK, _CHUNK)]],
        bufs.at[b],
        gsems.at[b],
    )

  def wb(i, b):
    return pltpu.make_async_copy(
        bufs.at[b],
        out_hbm.at[pl.ds(base + i * _CHUNK, _CHUNK)],
        wsems.at[b],
    )

  def slot(i, b, do_a, do_b):
    if do_a:
      wb(i - 1, (b - 1) % _NBUF).wait()
    if do_b:
      gather(i + _NBUF - 1, (b - 1) % _NBUF).start()
    gather(i, b).wait()
    wb(i, b).start()

  for b in range(_NBUF):
    gather(b, b).start()

  slot(0, 0, False, False)
  for b in range(1, _NBUF):
    slot(b, b, True, True)

  def body(g, _):
    i0 = g * _NBUF
    for b in range(_NBUF):
      slot(i0 + b, b, True, True)
    return 0

  lax.fori_loop(1, _N_CHUNKS // _NBUF - 1, body, 0)

  i0 = _N_CHUNKS - _NBUF
  slot(i0, 0, True, True)
  for b in range(1, _NBUF):
    slot(i0 + b, b, True, False)

  wb(_N_CHUNKS - 1, _NBUF - 1).wait()


def kernel(x, weight):
  wt = jnp.transpose(weight)
  w4 = _repack_table(wt)
  w_row = w4.reshape(_NLINES * 4, _DIM)
  out = _emb(x.reshape(-1), w_row)
  return out.reshape(_B, _J, _DIM)
